# Initial kernel scaffold; baseline (speedup 1.0000x reference)
import jax
import jax.numpy as jnp
from jax.experimental import pallas as pl


def kernel(x, edge_index, W1, b1, W2, b2):
    raise NotImplementedError("write your pallas kernel here")


# speedup vs baseline: 5.5283x; 5.5283x over previous
import functools

import jax
import jax.numpy as jnp
from jax import lax
from jax.experimental import pallas as pl
from jax.experimental.pallas import tpu as pltpu
from jax.experimental.pallas import tpu_sc as plsc

MIN_NORM = 1e-15
EPS = 4e-3

N_K = 10000
E_K = 320000
D_K = 128

NC = 2
NS = 16
NW = NC * NS
N_PAD = 10240
E_PAD = 327680
EPW = E_PAD // NW
CH = 128
NCH = EPW // CH
RPS = N_PAD // NS

R_TC = 256
TC_GRID = N_PAD // R_TC


def _rnorm(x):
    return jnp.clip(jnp.sqrt(jnp.sum(x * x, axis=-1, keepdims=True)),
                    MIN_NORM, None)


def _artanh(x):
    x = jnp.clip(x, -1.0 + 1e-7, 1.0 - 1e-7)
    return 0.5 * (jnp.log1p(x) - jnp.log1p(-x))


def _proj(x):
    norm = _rnorm(x)
    maxnorm = 1.0 - EPS
    return jnp.where(norm > maxnorm, x / norm * maxnorm, x)


def _expmap0(u):
    un = _rnorm(u)
    return jnp.tanh(un) * u / un


def _logmap0(p):
    pn = _rnorm(p)
    return p / pn * _artanh(pn)


def _mobius_add(x, y):
    x2 = jnp.sum(x * x, -1, keepdims=True)
    y2 = jnp.sum(y * y, -1, keepdims=True)
    xy = jnp.sum(x * y, -1, keepdims=True)
    num = (1.0 + 2.0 * xy + y2) * x + (1.0 - x2) * y
    denom = 1.0 + 2.0 * xy + x2 * y2
    return num / jnp.clip(denom, MIN_NORM, None)


def _mobius_matvec(W, x):
    xn = _rnorm(x)
    mx = lax.dot_general(x, W, (((1,), (1,)), ((), ())),
                         preferred_element_type=jnp.float32)
    mxn = _rnorm(mx)
    res = jnp.tanh(mxn / xn * _artanh(xn)) * mx / mxn
    cond = jnp.max(jnp.abs(mx), axis=-1, keepdims=True) == 0.0
    return jnp.where(cond, 0.0, res)


def _hyplinear(x_hyp, W, b):
    hyp_b = _proj(_expmap0(b))
    mv = _proj(_mobius_matvec(W, x_hyp))
    return _proj(_mobius_add(mv, hyp_b))


def _rdeg(dg):
    deg = jnp.clip(
        jnp.sum(dg, axis=-1, keepdims=True) * (1.0 / D_K),
        1.0, None)
    return lax.rsqrt(deg)


def _head_body(x_ref, w_ref, b_ref, dgo_ref, y_ref):
    x_hyp = _proj(_expmap0(x_ref[...]))
    h = _hyplinear(x_hyp, w_ref[...], b_ref[...])
    y_ref[...] = _logmap0(h) * _rdeg(dgo_ref[...])


def _mid_body(p_ref, dgi_ref, dgo_ref, w_ref, b_ref, y_ref):
    p = p_ref[...]
    agg = (p[0] + p[1]) * _rdeg(dgi_ref[...])
    h = _proj(_expmap0(agg))
    xt = jax.nn.relu(_logmap0(h))
    h1 = _proj(_expmap0(xt))
    h2 = _hyplinear(h1, w_ref[...], b_ref[...])
    y_ref[...] = _logmap0(h2) * _rdeg(dgo_ref[...])


def _tail_body(p_ref, dgi_ref, o_ref):
    p = p_ref[...]
    agg = (p[0] + p[1]) * _rdeg(dgi_ref[...])
    h = _proj(_expmap0(agg))
    xt = jax.nn.relu(_logmap0(h))
    o_ref[...] = _proj(_expmap0(xt))


_ROW = pl.BlockSpec((R_TC, D_K), lambda i: (i, 0))
_WMAT = pl.BlockSpec((D_K, D_K), lambda i: (0, 0))
_BVEC = pl.BlockSpec((1, D_K), lambda i: (0, 0))
_DEG = pl.BlockSpec((R_TC, D_K), lambda i: (i, 0))
_PART = pl.BlockSpec((2, R_TC, D_K), lambda i: (0, i, 0))
_OUT_SDS = jax.ShapeDtypeStruct((N_PAD, D_K), jnp.float32)


def _tc_head(x_pad, W1, b1, dgo):
    return pl.pallas_call(
        _head_body, grid=(TC_GRID,),
        in_specs=[_ROW, _WMAT, _BVEC, _DEG],
        out_specs=_ROW, out_shape=_OUT_SDS,
    )(x_pad, W1, b1.reshape(1, D_K), dgo)


def _tc_mid(p1, dgi, dgo, W2, b2):
    return pl.pallas_call(
        _mid_body, grid=(TC_GRID,),
        in_specs=[_PART, _DEG, _DEG, _WMAT, _BVEC],
        out_specs=_ROW, out_shape=_OUT_SDS,
    )(p1, dgi, dgo, W2, b2.reshape(1, D_K))


def _tc_tail(p2, dgi):
    return pl.pallas_call(
        _tail_body, grid=(TC_GRID,),
        in_specs=[_PART, _DEG],
        out_specs=_ROW, out_shape=_OUT_SDS,
    )(p2, dgi)


def _sc_mesh():
    return plsc.VectorSubcoreMesh(core_axis_name="c", subcore_axis_name="s")


def _sc_degrees(src, dst):
    EPS_DEG = E_PAD // NS
    NCH_DEG = EPS_DEG // CH

    @functools.partial(
        pl.kernel,
        out_type=(jax.ShapeDtypeStruct((N_PAD, D_K), jnp.float32),
                  jax.ShapeDtypeStruct((N_PAD, D_K), jnp.float32)),
        mesh=_sc_mesh(),
        scratch_types=[
            pltpu.VMEM((CH,), jnp.int32),
            pltpu.VMEM((CH, D_K), jnp.float32),
            pltpu.VMEM_SHARED((N_PAD, D_K), jnp.float32),
        ],
    )
    def k(sd_hbm, ones_hbm, zeros_hbm, dgo_hbm, dgi_hbm, idx, ones_v, acc):
        c = lax.axis_index("c")
        s = lax.axis_index("s")
        base = c * E_PAD + s * EPS_DEG
        rs = s * RPS
        pltpu.sync_copy(ones_hbm, ones_v)
        pltpu.sync_copy(zeros_hbm, acc.at[pl.ds(rs, RPS)])
        plsc.subcore_barrier()

        @pl.loop(0, NCH_DEG)
        def _(g):
            pltpu.sync_copy(sd_hbm.at[pl.ds(base + g * CH, CH)], idx)
            pltpu.sync_copy(ones_v, acc.at[idx], add=True)

        plsc.subcore_barrier()

        @pl.when(c == 0)
        def _():
            pltpu.sync_copy(acc.at[pl.ds(rs, RPS)], dgo_hbm.at[pl.ds(rs, RPS)])

        @pl.when(c == 1)
        def _():
            pltpu.sync_copy(acc.at[pl.ds(rs, RPS)], dgi_hbm.at[pl.ds(rs, RPS)])

    sd = jnp.concatenate([src, dst])
    return k(sd, jnp.ones((CH, D_K), jnp.float32),
             jnp.zeros((RPS, D_K), jnp.float32))


def _sc_aggregate(y, src, dst):
    @functools.partial(
        pl.kernel,
        out_type=jax.ShapeDtypeStruct((NC * N_PAD, D_K), jnp.float32),
        mesh=_sc_mesh(),
        scratch_types=[
            pltpu.VMEM((CH,), jnp.int32),
            pltpu.VMEM((CH,), jnp.int32),
            pltpu.VMEM((CH, D_K), jnp.float32),
            pltpu.VMEM_SHARED((N_PAD, D_K), jnp.float32),
            pltpu.SemaphoreType.DMA,
        ],
    )
    def k(y_hbm, src_hbm, dst_hbm, zeros_hbm, out_hbm, sidx, didx, rows,
          acc, sem):
        c = lax.axis_index("c")
        s = lax.axis_index("s")
        base = (c * NS + s) * EPW
        rs = s * RPS
        pltpu.sync_copy(zeros_hbm, acc.at[pl.ds(rs, RPS)])
        plsc.subcore_barrier()

        @pl.loop(0, NCH)
        def _(g):
            off = base + g * CH
            pltpu.sync_copy(src_hbm.at[pl.ds(off, CH)], sidx)
            pltpu.async_copy(y_hbm.at[sidx], rows, sem).wait()
            pltpu.sync_copy(dst_hbm.at[pl.ds(off, CH)], didx)
            pltpu.sync_copy(rows, acc.at[didx], add=True)

        plsc.subcore_barrier()
        wo = c * N_PAD + rs
        pltpu.sync_copy(acc.at[pl.ds(rs, RPS)], out_hbm.at[pl.ds(wo, RPS)])

    return k(y, src, dst,
             jnp.zeros((RPS, D_K), jnp.float32)).reshape(NC, N_PAD, D_K)


@jax.jit
def kernel(x, edge_index, W1, b1, W2, b2):
    x_pad = jnp.pad(x, ((0, N_PAD - N_K), (0, 0)))
    ei = jnp.pad(edge_index, ((0, 0), (0, E_PAD - E_K)),
                 constant_values=N_K)
    src = ei[0]
    dst = ei[1]

    dgo, dgi = _sc_degrees(src, dst)
    y1 = _tc_head(x_pad, W1, b1, dgo)
    p1 = _sc_aggregate(y1, src, dst)
    y2 = _tc_mid(p1, dgi, dgo, W2, b2)
    p2 = _sc_aggregate(y2, src, dst)
    out = _tc_tail(p2, dgi)
    return out[:N_K]


# speedup vs baseline: 13.2915x; 2.4043x over previous
import functools

import jax
import jax.numpy as jnp
from jax import lax
from jax.experimental import pallas as pl
from jax.experimental.pallas import tpu as pltpu
from jax.experimental.pallas import tpu_sc as plsc

MIN_NORM = 1e-15
EPS = 4e-3

N_K = 10000
E_K = 320000
D_K = 128

NC = 2
NS = 16
NW = NC * NS
N_PAD = 10240
E_PAD = 327680
EPW = E_PAD // NW
CH = 128
NCH = EPW // CH
RPS = N_PAD // NS

R_TC = 256
TC_GRID = N_PAD // R_TC


def _rnorm(x):
    return jnp.clip(jnp.sqrt(jnp.sum(x * x, axis=-1, keepdims=True)),
                    MIN_NORM, None)


def _artanh(x):
    x = jnp.clip(x, -1.0 + 1e-7, 1.0 - 1e-7)
    return 0.5 * (jnp.log1p(x) - jnp.log1p(-x))


def _proj(x):
    norm = _rnorm(x)
    maxnorm = 1.0 - EPS
    return jnp.where(norm > maxnorm, x / norm * maxnorm, x)


def _expmap0(u):
    un = _rnorm(u)
    return jnp.tanh(un) * u / un


def _logmap0(p):
    pn = _rnorm(p)
    return p / pn * _artanh(pn)


def _mobius_add(x, y):
    x2 = jnp.sum(x * x, -1, keepdims=True)
    y2 = jnp.sum(y * y, -1, keepdims=True)
    xy = jnp.sum(x * y, -1, keepdims=True)
    num = (1.0 + 2.0 * xy + y2) * x + (1.0 - x2) * y
    denom = 1.0 + 2.0 * xy + x2 * y2
    return num / jnp.clip(denom, MIN_NORM, None)


def _mobius_matvec(W, x):
    xn = _rnorm(x)
    mx = lax.dot_general(x, W, (((1,), (1,)), ((), ())),
                         preferred_element_type=jnp.float32)
    mxn = _rnorm(mx)
    res = jnp.tanh(mxn / xn * _artanh(xn)) * mx / mxn
    cond = jnp.max(jnp.abs(mx), axis=-1, keepdims=True) == 0.0
    return jnp.where(cond, 0.0, res)


def _hyplinear(x_hyp, W, b):
    hyp_b = _proj(_expmap0(b))
    mv = _proj(_mobius_matvec(W, x_hyp))
    return _proj(_mobius_add(mv, hyp_b))


def _rdeg(dg):
    deg = jnp.clip(
        jnp.sum(dg, axis=-1, keepdims=True) * (1.0 / D_K),
        1.0, None)
    return lax.rsqrt(deg)


def _head_body(x_ref, w_ref, b_ref, dgo_ref, y_ref):
    x_hyp = _proj(_expmap0(x_ref[...]))
    h = _hyplinear(x_hyp, w_ref[...], b_ref[...])
    y_ref[...] = _logmap0(h) * _rdeg(dgo_ref[...])


def _mid_body(p_ref, dgi_ref, dgo_ref, w_ref, b_ref, y_ref):
    p = p_ref[...]
    agg = (p[0] + p[1]) * _rdeg(dgi_ref[...])
    h = _proj(_expmap0(agg))
    xt = jax.nn.relu(_logmap0(h))
    h1 = _proj(_expmap0(xt))
    h2 = _hyplinear(h1, w_ref[...], b_ref[...])
    y_ref[...] = _logmap0(h2) * _rdeg(dgo_ref[...])


def _tail_body(p_ref, dgi_ref, o_ref):
    p = p_ref[...]
    agg = (p[0] + p[1]) * _rdeg(dgi_ref[...])
    h = _proj(_expmap0(agg))
    xt = jax.nn.relu(_logmap0(h))
    o_ref[...] = _proj(_expmap0(xt))


_ROW = pl.BlockSpec((R_TC, D_K), lambda i: (i, 0))
_WMAT = pl.BlockSpec((D_K, D_K), lambda i: (0, 0))
_BVEC = pl.BlockSpec((1, D_K), lambda i: (0, 0))
_DEG = pl.BlockSpec((R_TC, D_K), lambda i: (i, 0))
_PART = pl.BlockSpec((2, R_TC, D_K), lambda i: (0, i, 0))
_OUT_SDS = jax.ShapeDtypeStruct((N_PAD, D_K), jnp.float32)


def _tc_head(x_pad, W1, b1, dgo):
    return pl.pallas_call(
        _head_body, grid=(TC_GRID,),
        in_specs=[_ROW, _WMAT, _BVEC, _DEG],
        out_specs=_ROW, out_shape=_OUT_SDS,
    )(x_pad, W1, b1.reshape(1, D_K), dgo)


def _tc_mid(p1, dgi, dgo, W2, b2):
    return pl.pallas_call(
        _mid_body, grid=(TC_GRID,),
        in_specs=[_PART, _DEG, _DEG, _WMAT, _BVEC],
        out_specs=_ROW, out_shape=_OUT_SDS,
    )(p1, dgi, dgo, W2, b2.reshape(1, D_K))


def _tc_tail(p2, dgi):
    return pl.pallas_call(
        _tail_body, grid=(TC_GRID,),
        in_specs=[_PART, _DEG],
        out_specs=_ROW, out_shape=_OUT_SDS,
    )(p2, dgi)


def _sc_mesh():
    return plsc.VectorSubcoreMesh(core_axis_name="c", subcore_axis_name="s")


def _sc_degrees(src, dst):
    EPS_DEG = E_PAD // NS
    NCH_DEG = EPS_DEG // CH

    @functools.partial(
        pl.kernel,
        out_type=(jax.ShapeDtypeStruct((N_PAD, D_K), jnp.float32),
                  jax.ShapeDtypeStruct((N_PAD, D_K), jnp.float32)),
        mesh=_sc_mesh(),
        scratch_types=[
            pltpu.VMEM((CH,), jnp.int32),
            pltpu.VMEM((CH,), jnp.int32),
            pltpu.VMEM((CH, D_K), jnp.float32),
            pltpu.VMEM_SHARED((N_PAD, D_K), jnp.float32),
            pltpu.SemaphoreType.DMA,
            pltpu.SemaphoreType.DMA,
        ],
    )
    def k(sd_hbm, ones_hbm, zeros_hbm, dgo_hbm, dgi_hbm, idx0, idx1,
          ones_v, acc, sem0, sem1):
        c = lax.axis_index("c")
        s = lax.axis_index("s")
        base = c * E_PAD + s * EPS_DEG
        rs = s * RPS
        pltpu.sync_copy(ones_hbm, ones_v)
        pltpu.sync_copy(zeros_hbm, acc.at[pl.ds(rs, RPS)])
        plsc.subcore_barrier()

        def scat(off, idx, sem):
            pltpu.sync_copy(sd_hbm.at[pl.ds(off, CH)], idx)
            pltpu.async_copy(ones_v, acc.at[idx], sem, add=True)

        def drain(idx, sem):
            pltpu.make_async_copy(ones_v, acc.at[idx], sem).wait()

        scat(base, idx0, sem0)
        scat(base + CH, idx1, sem1)

        @pl.loop(1, NCH_DEG // 2)
        def _(i):
            off = base + i * (2 * CH)
            drain(idx0, sem0)
            scat(off, idx0, sem0)
            drain(idx1, sem1)
            scat(off + CH, idx1, sem1)

        drain(idx0, sem0)
        drain(idx1, sem1)
        plsc.subcore_barrier()

        @pl.when(c == 0)
        def _():
            pltpu.sync_copy(acc.at[pl.ds(rs, RPS)], dgo_hbm.at[pl.ds(rs, RPS)])

        @pl.when(c == 1)
        def _():
            pltpu.sync_copy(acc.at[pl.ds(rs, RPS)], dgi_hbm.at[pl.ds(rs, RPS)])

    sd = jnp.concatenate([src, dst])
    return k(sd, jnp.ones((CH, D_K), jnp.float32),
             jnp.zeros((RPS, D_K), jnp.float32))


def _sc_aggregate(y, src, dst):
    @functools.partial(
        pl.kernel,
        out_type=jax.ShapeDtypeStruct((NC * N_PAD, D_K), jnp.float32),
        mesh=_sc_mesh(),
        scratch_types=[
            pltpu.VMEM((CH,), jnp.int32),
            pltpu.VMEM((CH,), jnp.int32),
            pltpu.VMEM((CH,), jnp.int32),
            pltpu.VMEM((CH,), jnp.int32),
            pltpu.VMEM((CH, D_K), jnp.float32),
            pltpu.VMEM((CH, D_K), jnp.float32),
            pltpu.VMEM_SHARED((N_PAD, D_K), jnp.float32),
            pltpu.SemaphoreType.DMA,
            pltpu.SemaphoreType.DMA,
            pltpu.SemaphoreType.DMA,
            pltpu.SemaphoreType.DMA,
        ],
    )
    def k(y_hbm, src_hbm, dst_hbm, zeros_hbm, out_hbm, sidx0, sidx1,
          didx0, didx1, rows0, rows1, acc, gsem0, gsem1, ssem0, ssem1):
        c = lax.axis_index("c")
        s = lax.axis_index("s")
        base = (c * NS + s) * EPW
        rs = s * RPS
        pltpu.sync_copy(zeros_hbm, acc.at[pl.ds(rs, RPS)])
        plsc.subcore_barrier()

        def gather(off, sidx, rows, gsem):
            pltpu.sync_copy(src_hbm.at[pl.ds(off, CH)], sidx)
            return pltpu.async_copy(y_hbm.at[sidx], rows, gsem)

        def scatter(off, didx, rows, ssem):
            pltpu.sync_copy(dst_hbm.at[pl.ds(off, CH)], didx)
            pltpu.async_copy(rows, acc.at[didx], ssem, add=True)

        def drain(didx, rows, ssem):
            pltpu.make_async_copy(rows, acc.at[didx], ssem).wait()

        g0 = gather(base, sidx0, rows0, gsem0)
        g1 = gather(base + CH, sidx1, rows1, gsem1)
        g0.wait()
        scatter(base, didx0, rows0, ssem0)
        g1.wait()
        scatter(base + CH, didx1, rows1, ssem1)

        @pl.loop(1, NCH // 2)
        def _(i):
            off = base + i * (2 * CH)
            drain(didx0, rows0, ssem0)
            ga = gather(off, sidx0, rows0, gsem0)
            drain(didx1, rows1, ssem1)
            gb = gather(off + CH, sidx1, rows1, gsem1)
            ga.wait()
            scatter(off, didx0, rows0, ssem0)
            gb.wait()
            scatter(off + CH, didx1, rows1, ssem1)

        drain(didx0, rows0, ssem0)
        drain(didx1, rows1, ssem1)
        plsc.subcore_barrier()
        wo = c * N_PAD + rs
        pltpu.sync_copy(acc.at[pl.ds(rs, RPS)], out_hbm.at[pl.ds(wo, RPS)])

    return k(y, src, dst,
             jnp.zeros((RPS, D_K), jnp.float32)).reshape(NC, N_PAD, D_K)


@jax.jit
def kernel(x, edge_index, W1, b1, W2, b2):
    x_pad = jnp.pad(x, ((0, N_PAD - N_K), (0, 0)))
    pad_idx = N_K + jnp.arange(E_PAD - E_K, dtype=jnp.int32) % (N_PAD - N_K)
    src = jnp.concatenate([edge_index[0], pad_idx])
    dst = jnp.concatenate([edge_index[1], pad_idx])

    dgo, dgi = _sc_degrees(src, dst)
    y1 = _tc_head(x_pad, W1, b1, dgo)
    p1 = _sc_aggregate(y1, src, dst)
    y2 = _tc_mid(p1, dgi, dgo, W2, b2)
    p2 = _sc_aggregate(y2, src, dst)
    out = _tc_tail(p2, dgi)
    return out[:N_K]


# speedup vs baseline: 13.6486x; 1.0269x over previous
import functools

import jax
import jax.numpy as jnp
from jax import lax
from jax.experimental import pallas as pl
from jax.experimental.pallas import tpu as pltpu
from jax.experimental.pallas import tpu_sc as plsc

MIN_NORM = 1e-15
EPS = 4e-3

N_K = 10000
E_K = 320000
D_K = 128

NC = 2
NS = 16
NW = NC * NS
N_PAD = 10240
E_PAD = 327680
EPW = E_PAD // NW
CH = 128
NCH = EPW // CH
RPS = N_PAD // NS

D_W = 128
R_TC = 256
TC_GRID = N_PAD // R_TC


def _rnorm(x):
    return jnp.clip(jnp.sqrt(jnp.sum(x * x, axis=-1, keepdims=True)),
                    MIN_NORM, None)


def _artanh(x):
    x = jnp.clip(x, -1.0 + 1e-7, 1.0 - 1e-7)
    return 0.5 * (jnp.log1p(x) - jnp.log1p(-x))


def _proj(x):
    norm = _rnorm(x)
    maxnorm = 1.0 - EPS
    return jnp.where(norm > maxnorm, x / norm * maxnorm, x)


def _expmap0(u):
    un = _rnorm(u)
    return jnp.tanh(un) * u / un


def _logmap0(p):
    pn = _rnorm(p)
    return p / pn * _artanh(pn)


def _mobius_add(x, y):
    x2 = jnp.sum(x * x, -1, keepdims=True)
    y2 = jnp.sum(y * y, -1, keepdims=True)
    xy = jnp.sum(x * y, -1, keepdims=True)
    num = (1.0 + 2.0 * xy + y2) * x + (1.0 - x2) * y
    denom = 1.0 + 2.0 * xy + x2 * y2
    return num / jnp.clip(denom, MIN_NORM, None)


def _mobius_matvec(W, x):
    xn = _rnorm(x)
    mx = lax.dot_general(x, W, (((1,), (1,)), ((), ())),
                         preferred_element_type=jnp.float32)
    mxn = _rnorm(mx)
    res = jnp.tanh(mxn / xn * _artanh(xn)) * mx / mxn
    cond = jnp.max(jnp.abs(mx), axis=-1, keepdims=True) == 0.0
    return jnp.where(cond, 0.0, res)


def _hyplinear(x_hyp, W, b):
    hyp_b = _proj(_expmap0(b))
    mv = _proj(_mobius_matvec(W, x_hyp))
    return _proj(_mobius_add(mv, hyp_b))


def _rdeg(dg):
    deg = jnp.clip(
        jnp.sum(dg, axis=-1, keepdims=True) * (1.0 / D_W),
        1.0, None)
    return lax.rsqrt(deg)


def _head_body(x_ref, w_ref, b_ref, y_ref):
    x_hyp = _proj(_expmap0(x_ref[...]))
    h = _hyplinear(x_hyp, w_ref[...], b_ref[...])
    y_ref[...] = _logmap0(h)


def _scale_body(x_ref, dg_ref, y_ref):
    y_ref[...] = x_ref[...] * _rdeg(dg_ref[...])


def _mid_body(p_ref, dgi_ref, dgo_ref, w_ref, b_ref, y_ref):
    p = p_ref[...]
    agg = (p[0] + p[1]) * _rdeg(dgi_ref[...])
    h = _proj(_expmap0(agg))
    xt = jax.nn.relu(_logmap0(h))
    h1 = _proj(_expmap0(xt))
    h2 = _hyplinear(h1, w_ref[...], b_ref[...])
    y_ref[...] = _logmap0(h2) * _rdeg(dgo_ref[...])


def _tail_body(p_ref, dgi_ref, o_ref):
    p = p_ref[...]
    agg = (p[0] + p[1]) * _rdeg(dgi_ref[...])
    h = _proj(_expmap0(agg))
    xt = jax.nn.relu(_logmap0(h))
    o_ref[...] = _proj(_expmap0(xt))


_ROW = pl.BlockSpec((R_TC, D_K), lambda i: (i, 0))
_WMAT = pl.BlockSpec((D_K, D_K), lambda i: (0, 0))
_BVEC = pl.BlockSpec((1, D_K), lambda i: (0, 0))
_DEG = pl.BlockSpec((R_TC, D_W), lambda i: (i, 0))
_PART = pl.BlockSpec((2, R_TC, D_K), lambda i: (0, i, 0))
_OUT_SDS = jax.ShapeDtypeStruct((N_PAD, D_K), jnp.float32)


def _tc_head(x_pad, W1, b1):
    return pl.pallas_call(
        _head_body, grid=(TC_GRID,),
        in_specs=[_ROW, _WMAT, _BVEC],
        out_specs=_ROW, out_shape=_OUT_SDS,
    )(x_pad, W1, b1.reshape(1, D_K))


def _tc_scale(xt, dgo):
    return pl.pallas_call(
        _scale_body, grid=(TC_GRID,),
        in_specs=[_ROW, _DEG],
        out_specs=_ROW, out_shape=_OUT_SDS,
    )(xt, dgo)


def _tc_mid(p1, dgi, dgo, W2, b2):
    return pl.pallas_call(
        _mid_body, grid=(TC_GRID,),
        in_specs=[_PART, _DEG, _DEG, _WMAT, _BVEC],
        out_specs=_ROW, out_shape=_OUT_SDS,
    )(p1, dgi, dgo, W2, b2.reshape(1, D_K))


def _tc_tail(p2, dgi):
    return pl.pallas_call(
        _tail_body, grid=(TC_GRID,),
        in_specs=[_PART, _DEG],
        out_specs=_ROW, out_shape=_OUT_SDS,
    )(p2, dgi)


def _sc_mesh():
    return plsc.VectorSubcoreMesh(core_axis_name="c", subcore_axis_name="s")


def _sc_degrees(src, dst):
    EPS_DEG = E_PAD // NS
    NCH_DEG = EPS_DEG // CH

    @functools.partial(
        pl.kernel,
        out_type=(jax.ShapeDtypeStruct((N_PAD, D_W), jnp.float32),
                  jax.ShapeDtypeStruct((N_PAD, D_W), jnp.float32)),
        mesh=_sc_mesh(),
        scratch_types=[
            pltpu.VMEM((CH,), jnp.int32),
            pltpu.VMEM((CH,), jnp.int32),
            pltpu.VMEM((CH, D_W), jnp.float32),
            pltpu.VMEM_SHARED((N_PAD, D_W), jnp.float32),
            pltpu.SemaphoreType.DMA,
            pltpu.SemaphoreType.DMA,
        ],
    )
    def k(sd_hbm, ones_hbm, zeros_hbm, dgo_hbm, dgi_hbm, idx0, idx1,
          ones_v, acc, sem0, sem1):
        c = lax.axis_index("c")
        s = lax.axis_index("s")
        base = c * E_PAD + s * EPS_DEG
        rs = s * RPS
        pltpu.sync_copy(ones_hbm, ones_v)
        pltpu.sync_copy(zeros_hbm, acc.at[pl.ds(rs, RPS)])
        plsc.subcore_barrier()

        def scat(off, idx, sem):
            pltpu.sync_copy(sd_hbm.at[pl.ds(off, CH)], idx)
            pltpu.async_copy(ones_v, acc.at[idx], sem, add=True)

        def drain(idx, sem):
            pltpu.make_async_copy(ones_v, acc.at[idx], sem).wait()

        scat(base, idx0, sem0)
        scat(base + CH, idx1, sem1)

        @pl.loop(1, NCH_DEG // 2)
        def _(i):
            off = base + i * (2 * CH)
            drain(idx0, sem0)
            scat(off, idx0, sem0)
            drain(idx1, sem1)
            scat(off + CH, idx1, sem1)

        drain(idx0, sem0)
        drain(idx1, sem1)
        plsc.subcore_barrier()

        @pl.when(c == 0)
        def _():
            pltpu.sync_copy(acc.at[pl.ds(rs, RPS)], dgo_hbm.at[pl.ds(rs, RPS)])

        @pl.when(c == 1)
        def _():
            pltpu.sync_copy(acc.at[pl.ds(rs, RPS)], dgi_hbm.at[pl.ds(rs, RPS)])

    sd = jnp.concatenate([src, dst])
    return k(sd, jnp.ones((CH, D_W), jnp.float32),
             jnp.zeros((RPS, D_W), jnp.float32))


def _sc_aggregate(y, src, dst):
    @functools.partial(
        pl.kernel,
        out_type=jax.ShapeDtypeStruct((NC * N_PAD, D_K), jnp.float32),
        mesh=_sc_mesh(),
        scratch_types=[
            pltpu.VMEM((CH,), jnp.int32),
            pltpu.VMEM((CH,), jnp.int32),
            pltpu.VMEM((CH,), jnp.int32),
            pltpu.VMEM((CH,), jnp.int32),
            pltpu.VMEM((CH, D_K), jnp.float32),
            pltpu.VMEM((CH, D_K), jnp.float32),
            pltpu.VMEM_SHARED((N_PAD, D_K), jnp.float32),
            pltpu.SemaphoreType.DMA,
            pltpu.SemaphoreType.DMA,
            pltpu.SemaphoreType.DMA,
            pltpu.SemaphoreType.DMA,
        ],
    )
    def k(y_hbm, src_hbm, dst_hbm, zeros_hbm, out_hbm, sidx0, sidx1,
          didx0, didx1, rows0, rows1, acc, gsem0, gsem1, ssem0, ssem1):
        c = lax.axis_index("c")
        s = lax.axis_index("s")
        base = (c * NS + s) * EPW
        rs = s * RPS
        pltpu.sync_copy(zeros_hbm, acc.at[pl.ds(rs, RPS)])
        plsc.subcore_barrier()

        def gather(off, sidx, rows, gsem):
            pltpu.sync_copy(src_hbm.at[pl.ds(off, CH)], sidx)
            return pltpu.async_copy(y_hbm.at[sidx], rows, gsem)

        def scatter(off, didx, rows, ssem):
            pltpu.sync_copy(dst_hbm.at[pl.ds(off, CH)], didx)
            pltpu.async_copy(rows, acc.at[didx], ssem, add=True)

        def drain(didx, rows, ssem):
            pltpu.make_async_copy(rows, acc.at[didx], ssem).wait()

        g0 = gather(base, sidx0, rows0, gsem0)
        g1 = gather(base + CH, sidx1, rows1, gsem1)
        g0.wait()
        scatter(base, didx0, rows0, ssem0)
        g1.wait()
        scatter(base + CH, didx1, rows1, ssem1)

        @pl.loop(1, NCH // 2)
        def _(i):
            off = base + i * (2 * CH)
            drain(didx0, rows0, ssem0)
            ga = gather(off, sidx0, rows0, gsem0)
            drain(didx1, rows1, ssem1)
            gb = gather(off + CH, sidx1, rows1, gsem1)
            ga.wait()
            scatter(off, didx0, rows0, ssem0)
            gb.wait()
            scatter(off + CH, didx1, rows1, ssem1)

        drain(didx0, rows0, ssem0)
        drain(didx1, rows1, ssem1)
        plsc.subcore_barrier()
        wo = c * N_PAD + rs
        pltpu.sync_copy(acc.at[pl.ds(rs, RPS)], out_hbm.at[pl.ds(wo, RPS)])

    return k(y, src, dst,
             jnp.zeros((RPS, D_K), jnp.float32)).reshape(NC, N_PAD, D_K)


@jax.jit
def kernel(x, edge_index, W1, b1, W2, b2):
    x_pad = jnp.pad(x, ((0, N_PAD - N_K), (0, 0)))
    pad_idx = N_K + jnp.arange(E_PAD - E_K, dtype=jnp.int32) % (N_PAD - N_K)
    src = jnp.concatenate([edge_index[0], pad_idx])
    dst = jnp.concatenate([edge_index[1], pad_idx])

    xt1 = _tc_head(x_pad, W1, b1)
    dgo, dgi = _sc_degrees(src, dst)
    y1 = _tc_scale(xt1, dgo)
    p1 = _sc_aggregate(y1, src, dst)
    y2 = _tc_mid(p1, dgi, dgo, W2, b2)
    p2 = _sc_aggregate(y2, src, dst)
    out = _tc_tail(p2, dgi)
    return out[:N_K]


# speedup vs baseline: 14.2488x; 1.0440x over previous
import functools

import jax
import jax.numpy as jnp
from jax import lax
from jax.experimental import pallas as pl
from jax.experimental.pallas import tpu as pltpu
from jax.experimental.pallas import tpu_sc as plsc

MIN_NORM = 1e-15
EPS = 4e-3

N_K = 10000
E_K = 320000
D_K = 128

NC = 2
NS = 16
NW = NC * NS
N_PAD = 10240
E_PAD = 327680
EPW = E_PAD // NW
CH = 128
NCH = EPW // CH
RPS = N_PAD // NS

D_W = 128
R_TC = 256
TC_GRID = N_PAD // R_TC


def _rnorm(x):
    return jnp.clip(jnp.sqrt(jnp.sum(x * x, axis=-1, keepdims=True)),
                    MIN_NORM, None)


def _artanh(x):
    x = jnp.clip(x, -1.0 + 1e-7, 1.0 - 1e-7)
    return 0.5 * (jnp.log1p(x) - jnp.log1p(-x))


def _proj(x):
    norm = _rnorm(x)
    maxnorm = 1.0 - EPS
    return jnp.where(norm > maxnorm, x / norm * maxnorm, x)


def _expmap0(u):
    un = _rnorm(u)
    return jnp.tanh(un) * u / un


def _logmap0(p):
    pn = _rnorm(p)
    return p / pn * _artanh(pn)


def _mobius_add(x, y):
    x2 = jnp.sum(x * x, -1, keepdims=True)
    y2 = jnp.sum(y * y, -1, keepdims=True)
    xy = jnp.sum(x * y, -1, keepdims=True)
    num = (1.0 + 2.0 * xy + y2) * x + (1.0 - x2) * y
    denom = 1.0 + 2.0 * xy + x2 * y2
    return num / jnp.clip(denom, MIN_NORM, None)


def _mobius_matvec(W, x):
    xn = _rnorm(x)
    mx = lax.dot_general(x, W, (((1,), (1,)), ((), ())),
                         preferred_element_type=jnp.float32)
    mxn = _rnorm(mx)
    res = jnp.tanh(mxn / xn * _artanh(xn)) * mx / mxn
    cond = jnp.max(jnp.abs(mx), axis=-1, keepdims=True) == 0.0
    return jnp.where(cond, 0.0, res)


def _hyplinear(x_hyp, W, b):
    hyp_b = _proj(_expmap0(b))
    mv = _proj(_mobius_matvec(W, x_hyp))
    return _proj(_mobius_add(mv, hyp_b))


def _rdeg(dg):
    deg = jnp.clip(
        jnp.sum(dg, axis=-1, keepdims=True) * (1.0 / D_W),
        1.0, None)
    return lax.rsqrt(deg)


def _head_body(x_ref, w_ref, b_ref, y_ref):
    x_hyp = _proj(_expmap0(x_ref[...]))
    h = _hyplinear(x_hyp, w_ref[...], b_ref[...])
    y_ref[...] = _logmap0(h)


def _scale_body(x_ref, dg_ref, y_ref):
    y_ref[...] = x_ref[...] * _rdeg(dg_ref[...])


def _mid_body(p_ref, dgi_ref, dgo_ref, w_ref, b_ref, y_ref):
    p = p_ref[...]
    agg = (p[0] + p[1]) * _rdeg(dgi_ref[...])
    h = _proj(_expmap0(agg))
    xt = jax.nn.relu(_logmap0(h))
    h1 = _proj(_expmap0(xt))
    h2 = _hyplinear(h1, w_ref[...], b_ref[...])
    y_ref[...] = _logmap0(h2) * _rdeg(dgo_ref[...])


def _tail_body(p_ref, dgi_ref, o_ref):
    p = p_ref[...]
    agg = (p[0] + p[1]) * _rdeg(dgi_ref[...])
    h = _proj(_expmap0(agg))
    xt = jax.nn.relu(_logmap0(h))
    o_ref[...] = _proj(_expmap0(xt))


_ROW = pl.BlockSpec((R_TC, D_K), lambda i: (i, 0))
_WMAT = pl.BlockSpec((D_K, D_K), lambda i: (0, 0))
_BVEC = pl.BlockSpec((1, D_K), lambda i: (0, 0))
_DEG = pl.BlockSpec((R_TC, D_W), lambda i: (i, 0))
_PART = pl.BlockSpec((2, R_TC, D_K), lambda i: (0, i, 0))
_OUT_SDS = jax.ShapeDtypeStruct((N_PAD, D_K), jnp.float32)


def _tc_head(x_pad, W1, b1):
    return pl.pallas_call(
        _head_body, grid=(TC_GRID,),
        in_specs=[_ROW, _WMAT, _BVEC],
        out_specs=_ROW, out_shape=_OUT_SDS,
    )(x_pad, W1, b1.reshape(1, D_K))


def _tc_scale(xt, dgo):
    return pl.pallas_call(
        _scale_body, grid=(TC_GRID,),
        in_specs=[_ROW, _DEG],
        out_specs=_ROW, out_shape=_OUT_SDS,
    )(xt, dgo)


def _tc_mid(p1, dgi, dgo, W2, b2):
    return pl.pallas_call(
        _mid_body, grid=(TC_GRID,),
        in_specs=[_PART, _DEG, _DEG, _WMAT, _BVEC],
        out_specs=_ROW, out_shape=_OUT_SDS,
    )(p1, dgi, dgo, W2, b2.reshape(1, D_K))


def _tc_tail(p2, dgi):
    return pl.pallas_call(
        _tail_body, grid=(TC_GRID,),
        in_specs=[_PART, _DEG],
        out_specs=_ROW, out_shape=_OUT_SDS,
    )(p2, dgi)


def _sc_mesh():
    return plsc.VectorSubcoreMesh(core_axis_name="c", subcore_axis_name="s")


def _sc_degrees(src, dst):
    EPS_DEG = E_PAD // NS
    NCH_DEG = EPS_DEG // CH

    @functools.partial(
        pl.kernel,
        out_type=(jax.ShapeDtypeStruct((N_PAD, D_W), jnp.float32),
                  jax.ShapeDtypeStruct((N_PAD, D_W), jnp.float32)),
        mesh=_sc_mesh(),
        scratch_types=[
            pltpu.VMEM((CH,), jnp.int32),
            pltpu.VMEM((CH,), jnp.int32),
            pltpu.VMEM((CH, D_W), jnp.float32),
            pltpu.VMEM_SHARED((N_PAD, D_W), jnp.float32),
            pltpu.SemaphoreType.DMA,
            pltpu.SemaphoreType.DMA,
        ],
    )
    def k(sd_hbm, ones_hbm, zeros_hbm, dgo_hbm, dgi_hbm, idx0, idx1,
          ones_v, acc, sem0, sem1):
        c = lax.axis_index("c")
        s = lax.axis_index("s")
        base = c * E_PAD + s * EPS_DEG
        rs = s * RPS
        pltpu.sync_copy(ones_hbm, ones_v)
        pltpu.sync_copy(zeros_hbm, acc.at[pl.ds(rs, RPS)])
        plsc.subcore_barrier()

        def scat(off, idx, sem):
            pltpu.sync_copy(sd_hbm.at[pl.ds(off, CH)], idx)
            pltpu.async_copy(ones_v, acc.at[idx], sem, add=True)

        def drain(idx, sem):
            pltpu.make_async_copy(ones_v, acc.at[idx], sem).wait()

        scat(base, idx0, sem0)
        scat(base + CH, idx1, sem1)

        @pl.loop(1, NCH_DEG // 2)
        def _(i):
            off = base + i * (2 * CH)
            drain(idx0, sem0)
            scat(off, idx0, sem0)
            drain(idx1, sem1)
            scat(off + CH, idx1, sem1)

        drain(idx0, sem0)
        drain(idx1, sem1)
        plsc.subcore_barrier()

        @pl.when(c == 0)
        def _():
            pltpu.sync_copy(acc.at[pl.ds(rs, RPS)], dgo_hbm.at[pl.ds(rs, RPS)])

        @pl.when(c == 1)
        def _():
            pltpu.sync_copy(acc.at[pl.ds(rs, RPS)], dgi_hbm.at[pl.ds(rs, RPS)])

    sd = jnp.concatenate([src, dst])
    return k(sd, jnp.ones((CH, D_W), jnp.float32),
             jnp.zeros((RPS, D_W), jnp.float32))


def _sc_aggregate(y, src, dst):
    @functools.partial(
        pl.kernel,
        out_type=jax.ShapeDtypeStruct((NC * N_PAD, D_K), jnp.float32),
        mesh=_sc_mesh(),
        scratch_types=[
            pltpu.VMEM((EPW,), jnp.int32),
            pltpu.VMEM((CH,), jnp.int32),
            pltpu.VMEM((CH,), jnp.int32),
            pltpu.VMEM((CH, D_K), jnp.float32),
            pltpu.VMEM((CH, D_K), jnp.float32),
            pltpu.VMEM_SHARED((N_PAD, D_K), jnp.float32),
            pltpu.SemaphoreType.DMA,
            pltpu.SemaphoreType.DMA,
            pltpu.SemaphoreType.DMA,
            pltpu.SemaphoreType.DMA,
        ],
    )
    def k(y_hbm, src_hbm, dst_hbm, zeros_hbm, out_hbm, sall,
          didx0, didx1, rows0, rows1, acc, gsem0, gsem1, ssem0, ssem1):
        c = lax.axis_index("c")
        s = lax.axis_index("s")
        base = (c * NS + s) * EPW
        rs = s * RPS
        pltpu.sync_copy(src_hbm.at[pl.ds(base, EPW)], sall)
        pltpu.sync_copy(zeros_hbm, acc.at[pl.ds(rs, RPS)])
        plsc.subcore_barrier()

        def gather(g, rows, gsem):
            return pltpu.async_copy(
                y_hbm.at[sall.at[pl.ds(g * CH, CH)]], rows, gsem)

        def scatter(g, didx, rows, ssem):
            pltpu.sync_copy(dst_hbm.at[pl.ds(base + g * CH, CH)], didx)
            pltpu.async_copy(rows, acc.at[didx], ssem, add=True)

        def drain(didx, rows, ssem):
            pltpu.make_async_copy(rows, acc.at[didx], ssem).wait()

        g0 = gather(0, rows0, gsem0)
        g1 = gather(1, rows1, gsem1)
        g0.wait()
        scatter(0, didx0, rows0, ssem0)
        g1.wait()
        scatter(1, didx1, rows1, ssem1)

        @pl.loop(1, NCH // 2)
        def _(i):
            drain(didx0, rows0, ssem0)
            ga = gather(2 * i, rows0, gsem0)
            drain(didx1, rows1, ssem1)
            gb = gather(2 * i + 1, rows1, gsem1)
            ga.wait()
            scatter(2 * i, didx0, rows0, ssem0)
            gb.wait()
            scatter(2 * i + 1, didx1, rows1, ssem1)

        drain(didx0, rows0, ssem0)
        drain(didx1, rows1, ssem1)
        plsc.subcore_barrier()
        wo = c * N_PAD + rs
        pltpu.sync_copy(acc.at[pl.ds(rs, RPS)], out_hbm.at[pl.ds(wo, RPS)])

    return k(y, src, dst,
             jnp.zeros((RPS, D_K), jnp.float32)).reshape(NC, N_PAD, D_K)


@jax.jit
def kernel(x, edge_index, W1, b1, W2, b2):
    x_pad = jnp.pad(x, ((0, N_PAD - N_K), (0, 0)))
    pad_idx = N_K + jnp.arange(E_PAD - E_K, dtype=jnp.int32) % (N_PAD - N_K)
    src = jnp.concatenate([edge_index[0], pad_idx])
    dst = jnp.concatenate([edge_index[1], pad_idx])

    xt1 = _tc_head(x_pad, W1, b1)
    dgo, dgi = _sc_degrees(src, dst)
    y1 = _tc_scale(xt1, dgo)
    p1 = _sc_aggregate(y1, src, dst)
    y2 = _tc_mid(p1, dgi, dgo, W2, b2)
    p2 = _sc_aggregate(y2, src, dst)
    out = _tc_tail(p2, dgi)
    return out[:N_K]
